# baseline (speedup 1.0000x reference)
import jax
import jax.numpy as jnp
from jax.experimental import pallas as pl
from jax.experimental.pallas import tpu as pltpu

_GROUP = 12500
_B = 1000
_K = 13
_EPS = 1e-6


def _gn_kernel(h_ref, w_ref, b_ref, ms_ref, o_ref, data_ref, sums_ref, coef_ref):
    i = pl.program_id(0)
    t = pl.program_id(1)
    nseg = pl.num_programs(0) - 1
    slot = jax.lax.rem(i, 2)
    inv_n = 1.0 / _GROUP

    @pl.when(i < nseg)
    def _stats():
        x = h_ref[...]
        data_ref[slot, pl.ds(t * _B, _B), :] = x

        def _accum(xm):
            ps = jnp.sum(xm, axis=0, keepdims=True)
            pss = jnp.sum(xm * xm, axis=0, keepdims=True)

            @pl.when(t == 0)
            def _():
                sums_ref[slot, 0:1, :] = ps
                sums_ref[slot, 1:2, :] = pss

            @pl.when(t != 0)
            def _():
                sums_ref[slot, 0:1, :] += ps
                sums_ref[slot, 1:2, :] += pss

        @pl.when(t < _K - 1)
        def _():
            _accum(x)

        @pl.when(t == _K - 1)
        def _():
            rowid = jax.lax.broadcasted_iota(jnp.int32, (_B, 256), 0)
            valid = _GROUP - (_K - 1) * _B
            _accum(jnp.where(rowid < valid, x, 0.0))

    @pl.when((i > 0) & (t == 0))
    def _finalize():
        prev = 1 - slot
        s = sums_ref[prev, 0:1, :]
        ss = sums_ref[prev, 1:2, :]
        m = s * inv_n
        mm = m * ms_ref[...]
        var = ss * inv_n - (2.0 * m - mm) * mm
        inv_std = jax.lax.rsqrt(var + _EPS)
        a = w_ref[...] * inv_std
        coef_ref[0:1, :] = a
        coef_ref[1:2, :] = b_ref[...] - a * mm

    @pl.when(i > 0)
    def _emit():
        prev = 1 - slot
        x = data_ref[prev, pl.ds(t * _B, _B), :]
        o_ref[...] = x * coef_ref[0:1, :] + coef_ref[1:2, :]


def kernel(h, weight, bias, mean_scale):
    n_rows, d = h.shape
    batch = n_rows // _GROUP
    hf = h.astype(jnp.float32).reshape(batch, _GROUP, d)
    w2 = weight.astype(jnp.float32).reshape(1, d)
    b2 = bias.astype(jnp.float32).reshape(1, d)
    ms2 = mean_scale.astype(jnp.float32).reshape(1, d)

    last = batch - 1

    out = pl.pallas_call(
        _gn_kernel,
        grid=(batch + 1, _K),
        in_specs=[
            pl.BlockSpec(
                (None, _B, d),
                lambda i, t: (
                    jnp.minimum(i, last),
                    jnp.where(i < last + 1, t, _K - 1),
                    0,
                ),
            ),
            pl.BlockSpec((1, d), lambda i, t: (0, 0)),
            pl.BlockSpec((1, d), lambda i, t: (0, 0)),
            pl.BlockSpec((1, d), lambda i, t: (0, 0)),
        ],
        out_specs=pl.BlockSpec(
            (None, _B, d),
            lambda i, t: (
                jnp.maximum(i - 1, 0),
                jnp.where(i > 0, t, 0),
                0,
            ),
        ),
        out_shape=jax.ShapeDtypeStruct((batch, _GROUP, d), jnp.float32),
        scratch_shapes=[
            pltpu.VMEM((2, _K * _B, 256), jnp.float32),
            pltpu.VMEM((2, 2, 256), jnp.float32),
            pltpu.VMEM((2, 256), jnp.float32),
        ],
    )(hf, w2, b2, ms2)

    return out.reshape(n_rows, d).astype(h.dtype)


# speedup vs baseline: 2.3600x; 2.3600x over previous
import jax
import jax.numpy as jnp
from jax.experimental import pallas as pl
from jax.experimental.pallas import tpu as pltpu

_GROUP = 12500
_B = 1000
_PAIR = 2 * _GROUP
_K = _PAIR // _B
_TS = _GROUP // _B
_SPLIT = _GROUP - _TS * _B
_EPS = 1e-6


def _gn_kernel(h_ref, w_ref, b_ref, ms_ref, o_ref, slab_ref, sums_ref, coef_ref):
    i = pl.program_id(0)
    t = pl.program_id(1)
    npair = pl.num_programs(0) - 1
    slot = jax.lax.rem(i, 2)
    prev = 1 - slot
    inv_n = 1.0 / _GROUP

    @pl.when((i > 0) & (t == 0))
    def _finalize():
        def coefs(s, ss):
            m = s * inv_n
            mm = m * ms_ref[...]
            var = ss * inv_n - (2.0 * m - mm) * mm
            a = w_ref[...] * jax.lax.rsqrt(var + _EPS)
            return a, b_ref[...] - a * mm

        a0, c0 = coefs(sums_ref[prev, 0:1, :], sums_ref[prev, 1:2, :])
        a1, c1 = coefs(sums_ref[prev, 2:3, :], sums_ref[prev, 3:4, :])
        coef_ref[0:1, :] = a0
        coef_ref[1:2, :] = c0
        coef_ref[2:3, :] = a1
        coef_ref[3:4, :] = c1

    @pl.when(i > 0)
    def _emit():
        y = slab_ref[pl.ds(t * _B, _B), :]

        @pl.when(t < _TS)
        def _():
            o_ref[...] = y * coef_ref[0:1, :] + coef_ref[1:2, :]

        @pl.when(t == _TS)
        def _():
            rowid = jax.lax.broadcasted_iota(jnp.int32, (_B, 256), 0)
            o_ref[...] = jnp.where(
                rowid < _SPLIT,
                y * coef_ref[0:1, :] + coef_ref[1:2, :],
                y * coef_ref[2:3, :] + coef_ref[3:4, :],
            )

        @pl.when(t > _TS)
        def _():
            o_ref[...] = y * coef_ref[2:3, :] + coef_ref[3:4, :]

    @pl.when(i < npair)
    def _ingest():
        x = h_ref[...]
        slab_ref[pl.ds(t * _B, _B), :] = x

        def acc(base, xm, init):
            ps = jnp.sum(xm, axis=0, keepdims=True)
            pss = jnp.sum(xm * xm, axis=0, keepdims=True)
            if init:
                sums_ref[slot, base:base + 1, :] = ps
                sums_ref[slot, base + 1:base + 2, :] = pss
            else:
                sums_ref[slot, base:base + 1, :] += ps
                sums_ref[slot, base + 1:base + 2, :] += pss

        @pl.when(t == 0)
        def _():
            acc(0, x, True)
            sums_ref[slot, 2:4, :] = jnp.zeros((2, 256), jnp.float32)

        @pl.when((t > 0) & (t < _TS))
        def _():
            acc(0, x, False)

        @pl.when(t == _TS)
        def _():
            acc(0, x[:_SPLIT], False)
            acc(2, x[_SPLIT:], False)

        @pl.when(t > _TS)
        def _():
            acc(2, x, False)


def kernel(h, weight, bias, mean_scale):
    n_rows, d = h.shape
    npair = n_rows // _PAIR
    hf = h.astype(jnp.float32)
    w2 = weight.astype(jnp.float32).reshape(1, d)
    b2 = bias.astype(jnp.float32).reshape(1, d)
    ms2 = mean_scale.astype(jnp.float32).reshape(1, d)

    last_blk = n_rows // _B - 1

    out = pl.pallas_call(
        _gn_kernel,
        grid=(npair + 1, _K),
        in_specs=[
            pl.BlockSpec(
                (_B, d),
                lambda i, t: (
                    jnp.where(i < npair, jnp.minimum(i, npair - 1) * _K + t,
                              last_blk),
                    0,
                ),
            ),
            pl.BlockSpec((1, d), lambda i, t: (0, 0)),
            pl.BlockSpec((1, d), lambda i, t: (0, 0)),
            pl.BlockSpec((1, d), lambda i, t: (0, 0)),
        ],
        out_specs=pl.BlockSpec(
            (_B, d),
            lambda i, t: (jnp.maximum(i - 1, 0) * _K + jnp.where(i > 0, t, 0), 0),
        ),
        out_shape=jax.ShapeDtypeStruct((n_rows, d), jnp.float32),
        scratch_shapes=[
            pltpu.VMEM((_PAIR, 256), jnp.float32),
            pltpu.VMEM((2, 4, 256), jnp.float32),
            pltpu.VMEM((4, 256), jnp.float32),
        ],
    )(hf, w2, b2, ms2)

    return out.astype(h.dtype)


# speedup vs baseline: 2.4162x; 1.0238x over previous
import jax
import jax.numpy as jnp
from jax.experimental import pallas as pl
from jax.experimental.pallas import tpu as pltpu

_GROUP = 12500
_B = 1000
_LAG = 13
_RING = 16
_EPS = 1e-6


def _gn_kernel(h_ref, w_ref, b_ref, ms_ref, o_ref, slab_ref, sums_ref, coef_ref):
    s = pl.program_id(0)
    n_in = pl.num_programs(0) - _LAG
    inv_n = 1.0 / _GROUP

    @pl.when(s < n_in)
    def _ingest():
        x = h_ref[...]
        slot = jax.lax.rem(s, _RING)
        slab_ref[pl.ds(slot * _B, _B), :] = x
        pos = jax.lax.rem(s * _B, _GROUP)
        seg = jax.lax.div(s * _B, _GROUP)

        def psums(xm):
            return (jnp.sum(xm, axis=0, keepdims=True),
                    jnp.sum(xm * xm, axis=0, keepdims=True))

        @pl.when(pos == 0)
        def _():
            ps, pss = psums(x)
            sums_ref[pl.ds(2 * seg, 1), :] = ps
            sums_ref[pl.ds(2 * seg + 1, 1), :] = pss

        @pl.when((pos > 0) & (pos + _B <= _GROUP))
        def _():
            ps, pss = psums(x)
            sums_ref[pl.ds(2 * seg, 1), :] += ps
            sums_ref[pl.ds(2 * seg + 1, 1), :] += pss

        @pl.when(pos + _B > _GROUP)
        def _():
            split = _GROUP - (_GROUP // _B) * _B
            ps, pss = psums(x[:split])
            sums_ref[pl.ds(2 * seg, 1), :] += ps
            sums_ref[pl.ds(2 * seg + 1, 1), :] += pss
            ps2, pss2 = psums(x[split:])
            sums_ref[pl.ds(2 * seg + 2, 1), :] = ps2
            sums_ref[pl.ds(2 * seg + 3, 1), :] = pss2

    @pl.when(s >= _LAG)
    def _emit():
        e = s - _LAG
        pos = jax.lax.rem(e * _B, _GROUP)
        seg = jax.lax.div(e * _B, _GROUP)
        straddle = pos + _B > _GROUP

        def finalize(j):
            sm = sums_ref[pl.ds(2 * j, 1), :]
            ss = sums_ref[pl.ds(2 * j + 1, 1), :]
            m = sm * inv_n
            mm = m * ms_ref[...]
            var = ss * inv_n - (2.0 * m - mm) * mm
            a = w_ref[...] * jax.lax.rsqrt(var + _EPS)
            coef_ref[pl.ds(2 * j, 1), :] = a
            coef_ref[pl.ds(2 * j + 1, 1), :] = b_ref[...] - a * mm

        @pl.when(pos == 0)
        def _():
            finalize(seg)

        @pl.when(straddle)
        def _():
            finalize(seg + 1)

        slot = jax.lax.rem(e, _RING)
        y = slab_ref[pl.ds(slot * _B, _B), :]
        a0 = coef_ref[pl.ds(2 * seg, 1), :]
        c0 = coef_ref[pl.ds(2 * seg + 1, 1), :]

        @pl.when(jnp.logical_not(straddle))
        def _():
            o_ref[...] = y * a0 + c0

        @pl.when(straddle)
        def _():
            split = _GROUP - (_GROUP // _B) * _B
            rowid = jax.lax.broadcasted_iota(jnp.int32, (_B, 256), 0)
            a1 = coef_ref[pl.ds(2 * seg + 2, 1), :]
            c1 = coef_ref[pl.ds(2 * seg + 3, 1), :]
            o_ref[...] = jnp.where(rowid < split, y * a0 + c0, y * a1 + c1)


def kernel(h, weight, bias, mean_scale):
    n_rows, d = h.shape
    n_blk = n_rows // _B
    hf = h.astype(jnp.float32)
    w2 = weight.astype(jnp.float32).reshape(1, d)
    b2 = bias.astype(jnp.float32).reshape(1, d)
    ms2 = mean_scale.astype(jnp.float32).reshape(1, d)

    out = pl.pallas_call(
        _gn_kernel,
        grid=(n_blk + _LAG,),
        in_specs=[
            pl.BlockSpec((_B, d), lambda s: (jnp.minimum(s, n_blk - 1), 0)),
            pl.BlockSpec((1, d), lambda s: (0, 0)),
            pl.BlockSpec((1, d), lambda s: (0, 0)),
            pl.BlockSpec((1, d), lambda s: (0, 0)),
        ],
        out_specs=pl.BlockSpec(
            (_B, d), lambda s: (jnp.maximum(s - _LAG, 0), 0)
        ),
        out_shape=jax.ShapeDtypeStruct((n_rows, d), jnp.float32),
        scratch_shapes=[
            pltpu.VMEM((_RING * _B, 256), jnp.float32),
            pltpu.VMEM((16, 256), jnp.float32),
            pltpu.VMEM((16, 256), jnp.float32),
        ],
    )(hf, w2, b2, ms2)

    return out.astype(h.dtype)
